# baseline (speedup 1.0000x reference)
import functools

import jax
import jax.numpy as jnp
from jax import lax
from jax.experimental import pallas as pl
from jax.experimental.pallas import tpu as pltpu
from jax.experimental.pallas import tpu_sc as plsc

RADIUS2 = 4.0
NSAMPLE = 32
NCAND = 180
NCP = 192
M = 16384
CFEAT = 16
RV_H, RV_W, PPP = 64, 2048, 4

NCORES, NSUBC = 2, 16
NW = NCORES * NSUBC
QPW = M // NW
G = 16
NGRP = QPW // G
CH = 512
SH = 9
CMASK = CH - 1
NCH_CAND = (G * NCP) // CH
NCH_SEL = (G * NSAMPLE) // CH
OW = (3 + CFEAT) * NSAMPLE


def _splat(x, dtype=jnp.int32):
    return jnp.full((16,), x, dtype=dtype)


def _vgather(v, idx):
    return v.at[idx].get(mode="promise_in_bounds")


def _elem(buf, pos):
    return plsc.load_gather(buf, [_splat(pos >> SH), _splat(pos & CMASK)])


def _sc_body(q_h, c_h, rvf_h, xyzp_h, feat_h, out_h,
             qbuf, cbuf, eidx0, eidx1, cand0, cand1, cxyz0, cxyz1,
             sel, gfeat, gxyz, cnts, outb0, outb1, semb, semc, seme, semg):
    wid = lax.axis_index("s") * NCORES + lax.axis_index("c")
    qbase = wid * QPW
    qrow = wid * (QPW * 3 // CH)
    iota = jnp.arange(16, dtype=jnp.int32)

    pltpu.sync_copy(q_h.at[pl.ds(qrow, QPW * 3 // CH)], qbuf)
    pltpu.sync_copy(c_h.at[pl.ds(qrow, QPW * 3 // CH)], cbuf)

    eidxs = (eidx0, eidx1)
    cands = (cand0, cand1)
    cxyzs = (cxyz0, cxyz1)
    outbs = (outb0, outb1)

    def build_group(g, eidx):
        for i in range(G):
            lq = g * G + i
            rsp = _elem(cbuf, lq * 3 + 1) & jnp.int32(RV_H - 1)
            csp = _elem(cbuf, lq * 3 + 2) & jnp.int32(RV_W - 1)
            cells = []
            for jj in range(3):
                u = iota + 16 * jj
                oh = u // 9 - 2
                ow = 2 * (u % 9) - 8
                rr = jnp.clip(rsp + oh, 0, RV_H - 1)
                cc = (csp + ow) & jnp.int32(RV_W - 1)
                cells.append(rr * RV_W + cc)
            for jj2 in range(12):
                lidx = iota // 4 + 4 * (jj2 % 4)
                cv = _vgather(cells[jj2 // 4], lidx)
                ev = cv * PPP + (iota & 3)
                p = i * NCP + 16 * jj2
                eidx[p >> SH, pl.ds(p & CMASK, 16)] = ev

    def fire_rv(eidx, cand):
        for j in range(NCH_CAND):
            pltpu.make_async_copy(rvf_h.at[eidx.at[j]], cand.at[j],
                                  semb).start()

    def drain_rv(eidx, cand):
        for j in range(NCH_CAND):
            pltpu.make_async_copy(rvf_h.at[eidx.at[j]], cand.at[j],
                                  semb).wait()

    def fire_xyz(cand, cxyz):
        for j in range(NCH_CAND):
            pltpu.make_async_copy(xyzp_h.at[cand.at[j]], cxyz.at[j],
                                  semc).start()

    def drain_xyz(cand, cxyz):
        for j in range(NCH_CAND):
            pltpu.make_async_copy(xyzp_h.at[cand.at[j]], cxyz.at[j],
                                  semc).wait()

    z16i = jnp.zeros((16,), jnp.int32)
    for j in range(NCH_SEL):
        for h8 in range(CH // 16):
            sel[j, pl.ds(16 * h8, 16)] = z16i

    build_group(0, eidx0)
    fire_rv(eidx0, cand0)
    drain_rv(eidx0, cand0)
    fire_xyz(cand0, cxyz0)
    build_group(1, eidx1)
    fire_rv(eidx1, cand1)

    def pair_body(t, carry):
        for par in range(2):
            g = 2 * t + par
            eidx, cand, cxyz = eidxs[par], cands[par], cxyzs[par]
            neidx, ncand, ncxyz = (eidxs[1 - par], cands[1 - par],
                                   cxyzs[1 - par])

            drain_xyz(cand, cxyz)

            @pl.when(g < NGRP - 1)
            def _():
                drain_rv(neidx, ncand)
                fire_xyz(ncand, ncxyz)

            @pl.when(g < NGRP - 2)
            def _():
                build_group(g + 2, eidx)

            def select_body(i, bc):
                lq = g * G + i
                xq = _elem(qbuf, lq * 3)
                yq = _elem(qbuf, lq * 3 + 1)
                zq = _elem(qbuf, lq * 3 + 2)
                z16 = _splat(0)
                cnt = jnp.int32(0)
                for jj in range(12):
                    p = _splat(i * NCP + 16 * jj) + iota
                    pr, pc = p >> SH, p & CMASK
                    cd = plsc.load_gather(cand, [pr, pc])
                    x = plsc.load_gather(cxyz, [pr, pc, z16])
                    y = plsc.load_gather(cxyz, [pr, pc, z16 + 1])
                    z = plsc.load_gather(cxyz, [pr, pc, z16 + 2])
                    dx, dy, dz = x - xq, y - yq, z - zq
                    d2 = dx * dx + dy * dy + dz * dz
                    val = d2 < RADIUS2
                    if jj == 11:
                        val = val & (iota < (NCAND - 16 * 11))
                    vi = val.astype(jnp.int32)
                    pref = plsc.cumsum(vi)
                    rank = cnt + pref - 1
                    m = val & (rank < NSAMPLE)
                    sp = _splat(i * NSAMPLE) + rank
                    plsc.store_scatter(sel, [sp >> SH, sp & CMASK], cd, mask=m)
                    cnt = cnt + pref[15]
                sp0 = i * NSAMPLE
                fsv = plsc.load_gather(
                    sel, [_splat(sp0 >> SH), _splat(sp0 & CMASK)])
                for h in range(2):
                    k = iota + 16 * h
                    spk = sp0 + k
                    cur = plsc.load_gather(sel, [spk >> SH, spk & CMASK])
                    new = jnp.where(k < cnt, cur, fsv)
                    plsc.store_scatter(sel, [spk >> SH, spk & CMASK], new)
                cnts[i] = cnt
                @pl.when((i & (CH // NSAMPLE - 1)) == CH // NSAMPLE - 1)
                def _():
                    j = i >> 4
                    pltpu.make_async_copy(feat_h.at[sel.at[j]],
                                          gfeat.at[j], seme).start()
                    pltpu.make_async_copy(xyzp_h.at[sel.at[j]],
                                          gxyz.at[j], seme).start()
                return bc

            lax.fori_loop(0, G, select_body, 0)

            @pl.when(g < NGRP - 2)
            def _():
                fire_rv(eidx, cand)

            for j in range(NCH_SEL):
                pltpu.make_async_copy(feat_h.at[sel.at[j]], gfeat.at[j],
                                      seme).wait()
                pltpu.make_async_copy(xyzp_h.at[sel.at[j]], gxyz.at[j],
                                      seme).wait()

            outb = outbs[par]
            @pl.when(g >= 2)
            def _():
                pltpu.make_async_copy(
                    outb, out_h.at[pl.ds(qbase + (g - 2) * G, G)],
                    semg).wait()

            def out_body(i, bc):
                lq = g * G + i
                xq = _elem(qbuf, lq * 3)
                yq = _elem(qbuf, lq * 3 + 1)
                zq = _elem(qbuf, lq * 3 + 2)
                isp = _splat(0) + i
                qs = (xq, yq, zq)
                for h in range(2):
                    sp = _splat(i * NSAMPLE + 16 * h) + iota
                    sr, sc = sp >> SH, sp & CMASK
                    k = _splat(16 * h) + iota
                    for c in range(3 + CFEAT):
                        if c < 3:
                            v = plsc.load_gather(
                                gxyz, [sr, sc, _splat(c)]) - qs[c]
                        else:
                            v = plsc.load_gather(
                                gfeat, [sr, sc, _splat(c - 3)])
                        plsc.store_scatter(outb, [isp, _splat(c), k], v)

                @pl.when(cnts[i] == 0)
                def _():
                    z16f = jnp.zeros((16,), jnp.float32)
                    for h in range(2):
                        k = _splat(16 * h) + iota
                        for c in range(3 + CFEAT):
                            plsc.store_scatter(
                                outb, [isp, _splat(c), k], z16f)
                return bc

            lax.fori_loop(0, G, out_body, 0)

            pltpu.make_async_copy(
                outb, out_h.at[pl.ds(qbase + g * G, G)], semg).start()
        return carry

    lax.fori_loop(0, NGRP // 2, pair_body, 0)

    for gl in (NGRP - 2, NGRP - 1):
        pltpu.make_async_copy(
            outbs[gl % 2], out_h.at[pl.ds(qbase + gl * G, G)],
            semg).wait()


def _impl(xyz, features, query_rv_xyz, query_rv_coords, rv_map):
    xyzp = jnp.concatenate(
        [xyz, jnp.zeros((xyz.shape[0], 5), jnp.float32)], axis=1)
    rvf = rv_map.reshape(-1)
    qv = query_rv_xyz.reshape(M * 3 // CH, CH)
    cv = query_rv_coords.reshape(M * 3 // CH, CH)

    mesh = plsc.VectorSubcoreMesh(core_axis_name="c", subcore_axis_name="s",
                                  num_cores=NCORES, num_subcores=NSUBC)
    run = pl.kernel(
        _sc_body,
        out_type=jax.ShapeDtypeStruct((M, 3 + CFEAT, NSAMPLE),
                                      jnp.float32),
        mesh=mesh,
        compiler_params=pltpu.CompilerParams(use_tc_tiling_on_sc=False,
                                             needs_layout_passes=False),
        scratch_types=[
            pltpu.VMEM((QPW * 3 // CH, CH), jnp.float32),
            pltpu.VMEM((QPW * 3 // CH, CH), jnp.int32),
            pltpu.VMEM((NCH_CAND, CH), jnp.int32),
            pltpu.VMEM((NCH_CAND, CH), jnp.int32),
            pltpu.VMEM((NCH_CAND, CH), jnp.int32),
            pltpu.VMEM((NCH_CAND, CH), jnp.int32),
            pltpu.VMEM((NCH_CAND, CH, 8), jnp.float32),
            pltpu.VMEM((NCH_CAND, CH, 8), jnp.float32),
            pltpu.VMEM((NCH_SEL, CH), jnp.int32),
            pltpu.VMEM((NCH_SEL, CH, CFEAT), jnp.float32),
            pltpu.VMEM((NCH_SEL, CH, 8), jnp.float32),
            pltpu.SMEM((G,), jnp.int32),
            pltpu.VMEM((G, 3 + CFEAT, NSAMPLE), jnp.float32),
            pltpu.VMEM((G, 3 + CFEAT, NSAMPLE), jnp.float32),
            pltpu.SemaphoreType.DMA,
            pltpu.SemaphoreType.DMA,
            pltpu.SemaphoreType.DMA,
            pltpu.SemaphoreType.DMA,
        ],
    )
    return run(qv, cv, rvf, xyzp, features)


_impl.__name__ = "kernel"
_JIT = None


def kernel(xyz, features, query_rv_xyz, query_rv_coords, rv_map):
    global _JIT
    if _JIT is None:
        _JIT = jax.jit(_impl)
    return _JIT(xyz, features, query_rv_xyz, query_rv_coords, rv_map)


# speedup vs baseline: 1.0057x; 1.0057x over previous
import functools

import jax
import jax.numpy as jnp
from jax import lax
from jax.experimental import pallas as pl
from jax.experimental.pallas import tpu as pltpu
from jax.experimental.pallas import tpu_sc as plsc

RADIUS2 = 4.0
NSAMPLE = 32
NCAND = 180
NCP = 192
M = 16384
CFEAT = 16
RV_H, RV_W, PPP = 64, 2048, 4

NCORES, NSUBC = 2, 16
NW = NCORES * NSUBC
QPW = M // NW
G = 16
NGRP = QPW // G
CH = 128
SH = 7
CMASK = CH - 1
NCH_CAND = (G * NCP) // CH
NCH_SEL = (G * NSAMPLE) // CH
OW = (3 + CFEAT) * NSAMPLE


def _splat(x, dtype=jnp.int32):
    return jnp.full((16,), x, dtype=dtype)


def _vgather(v, idx):
    return v.at[idx].get(mode="promise_in_bounds")


def _elem(buf, pos):
    return plsc.load_gather(buf, [_splat(pos >> SH), _splat(pos & CMASK)])


def _sc_body(q_h, c_h, rvf_h, xyzp_h, feat_h, out_h,
             qbuf, cbuf, eidx0, eidx1, cand0, cand1, cxyz0, cxyz1,
             sel, gfeat, gxyz, cnts, outb0, outb1, semb, semc, seme, semg):
    wid = lax.axis_index("s") * NCORES + lax.axis_index("c")
    qbase = wid * QPW
    qrow = wid * (QPW * 3 // CH)
    iota = jnp.arange(16, dtype=jnp.int32)

    pltpu.sync_copy(q_h.at[pl.ds(qrow, QPW * 3 // CH)], qbuf)
    pltpu.sync_copy(c_h.at[pl.ds(qrow, QPW * 3 // CH)], cbuf)

    eidxs = (eidx0, eidx1)
    cands = (cand0, cand1)
    cxyzs = (cxyz0, cxyz1)
    outbs = (outb0, outb1)

    def build_group(g, eidx):
        for i in range(G):
            lq = g * G + i
            rsp = _elem(cbuf, lq * 3 + 1) & jnp.int32(RV_H - 1)
            csp = _elem(cbuf, lq * 3 + 2) & jnp.int32(RV_W - 1)
            cells = []
            for jj in range(3):
                u = iota + 16 * jj
                oh = u // 9 - 2
                ow = 2 * (u % 9) - 8
                rr = jnp.clip(rsp + oh, 0, RV_H - 1)
                cc = (csp + ow) & jnp.int32(RV_W - 1)
                cells.append(rr * RV_W + cc)
            for jj2 in range(12):
                lidx = iota // 4 + 4 * (jj2 % 4)
                cv = _vgather(cells[jj2 // 4], lidx)
                ev = cv * PPP + (iota & 3)
                p = i * NCP + 16 * jj2
                eidx[p >> SH, pl.ds(p & CMASK, 16)] = ev

    def fire_rv(eidx, cand):
        for j in range(NCH_CAND):
            pltpu.make_async_copy(rvf_h.at[eidx.at[j]], cand.at[j],
                                  semb).start()

    def drain_rv(eidx, cand):
        for j in range(NCH_CAND):
            pltpu.make_async_copy(rvf_h.at[eidx.at[j]], cand.at[j],
                                  semb).wait()

    def fire_xyz(cand, cxyz):
        for j in range(NCH_CAND):
            pltpu.make_async_copy(xyzp_h.at[cand.at[j]], cxyz.at[j],
                                  semc).start()

    def drain_xyz(cand, cxyz):
        for j in range(NCH_CAND):
            pltpu.make_async_copy(xyzp_h.at[cand.at[j]], cxyz.at[j],
                                  semc).wait()

    z16i = jnp.zeros((16,), jnp.int32)
    for j in range(NCH_SEL):
        for h8 in range(CH // 16):
            sel[j, pl.ds(16 * h8, 16)] = z16i

    build_group(0, eidx0)
    fire_rv(eidx0, cand0)
    drain_rv(eidx0, cand0)
    fire_xyz(cand0, cxyz0)
    build_group(1, eidx1)
    fire_rv(eidx1, cand1)

    def pair_body(t, carry):
        for par in range(2):
            g = 2 * t + par
            eidx, cand, cxyz = eidxs[par], cands[par], cxyzs[par]
            neidx, ncand, ncxyz = (eidxs[1 - par], cands[1 - par],
                                   cxyzs[1 - par])

            drain_xyz(cand, cxyz)

            @pl.when(g < NGRP - 1)
            def _():
                drain_rv(neidx, ncand)
                fire_xyz(ncand, ncxyz)

            @pl.when(g < NGRP - 2)
            def _():
                build_group(g + 2, eidx)

            @plsc.parallel_loop(0, G)
            def select_body(i):
                lq = g * G + i
                xq = _elem(qbuf, lq * 3)
                yq = _elem(qbuf, lq * 3 + 1)
                zq = _elem(qbuf, lq * 3 + 2)
                z16 = _splat(0)
                cnt = jnp.int32(0)
                for jj in range(12):
                    p = _splat(i * NCP + 16 * jj) + iota
                    pr, pc = p >> SH, p & CMASK
                    cd = plsc.load_gather(cand, [pr, pc])
                    x = plsc.load_gather(cxyz, [pr, pc, z16])
                    y = plsc.load_gather(cxyz, [pr, pc, z16 + 1])
                    z = plsc.load_gather(cxyz, [pr, pc, z16 + 2])
                    dx, dy, dz = x - xq, y - yq, z - zq
                    d2 = dx * dx + dy * dy + dz * dz
                    val = d2 < RADIUS2
                    if jj == 11:
                        val = val & (iota < (NCAND - 16 * 11))
                    vi = val.astype(jnp.int32)
                    pref = plsc.cumsum(vi)
                    rank = cnt + pref - 1
                    m = val & (rank < NSAMPLE)
                    sp = _splat(i * NSAMPLE) + rank
                    plsc.store_scatter(sel, [sp >> SH, sp & CMASK], cd, mask=m)
                    cnt = cnt + pref[15]
                sp0 = i * NSAMPLE
                fsv = plsc.load_gather(
                    sel, [_splat(sp0 >> SH), _splat(sp0 & CMASK)])
                for h in range(2):
                    k = iota + 16 * h
                    spk = sp0 + k
                    cur = plsc.load_gather(sel, [spk >> SH, spk & CMASK])
                    new = jnp.where(k < cnt, cur, fsv)
                    plsc.store_scatter(sel, [spk >> SH, spk & CMASK], new)
                cnts[i] = cnt

            for j in range(NCH_SEL):
                pltpu.make_async_copy(feat_h.at[sel.at[j]],
                                      gfeat.at[j], seme).start()
                pltpu.make_async_copy(xyzp_h.at[sel.at[j]],
                                      gxyz.at[j], seme).start()

            @pl.when(g < NGRP - 2)
            def _():
                fire_rv(eidx, cand)

            for j in range(NCH_SEL):
                pltpu.make_async_copy(feat_h.at[sel.at[j]], gfeat.at[j],
                                      seme).wait()
                pltpu.make_async_copy(xyzp_h.at[sel.at[j]], gxyz.at[j],
                                      seme).wait()

            outb = outbs[par]
            @pl.when(g >= 2)
            def _():
                pltpu.make_async_copy(
                    outb, out_h.at[pl.ds(qbase + (g - 2) * G, G)],
                    semg).wait()

            @plsc.parallel_loop(0, G)
            def out_body(i):
                lq = g * G + i
                xq = _elem(qbuf, lq * 3)
                yq = _elem(qbuf, lq * 3 + 1)
                zq = _elem(qbuf, lq * 3 + 2)
                isp = _splat(0) + i
                qs = (xq, yq, zq)
                for h in range(2):
                    sp = _splat(i * NSAMPLE + 16 * h) + iota
                    sr, sc = sp >> SH, sp & CMASK
                    k = _splat(16 * h) + iota
                    for c in range(3 + CFEAT):
                        if c < 3:
                            v = plsc.load_gather(
                                gxyz, [sr, sc, _splat(c)]) - qs[c]
                        else:
                            v = plsc.load_gather(
                                gfeat, [sr, sc, _splat(c - 3)])
                        plsc.store_scatter(outb, [isp, _splat(c), k], v)

                @pl.when(cnts[i] == 0)
                def _():
                    z16f = jnp.zeros((16,), jnp.float32)
                    for h in range(2):
                        k = _splat(16 * h) + iota
                        for c in range(3 + CFEAT):
                            plsc.store_scatter(
                                outb, [isp, _splat(c), k], z16f)

            pltpu.make_async_copy(
                outb, out_h.at[pl.ds(qbase + g * G, G)], semg).start()
        return carry

    lax.fori_loop(0, NGRP // 2, pair_body, 0)

    for gl in (NGRP - 2, NGRP - 1):
        pltpu.make_async_copy(
            outbs[gl % 2], out_h.at[pl.ds(qbase + gl * G, G)],
            semg).wait()


def _impl(xyz, features, query_rv_xyz, query_rv_coords, rv_map):
    xyzp = jnp.concatenate(
        [xyz, jnp.zeros((xyz.shape[0], 5), jnp.float32)], axis=1)
    rvf = rv_map.reshape(-1)
    qv = query_rv_xyz.reshape(M * 3 // CH, CH)
    cv = query_rv_coords.reshape(M * 3 // CH, CH)

    mesh = plsc.VectorSubcoreMesh(core_axis_name="c", subcore_axis_name="s",
                                  num_cores=NCORES, num_subcores=NSUBC)
    run = pl.kernel(
        _sc_body,
        out_type=jax.ShapeDtypeStruct((M, 3 + CFEAT, NSAMPLE),
                                      jnp.float32),
        mesh=mesh,
        compiler_params=pltpu.CompilerParams(use_tc_tiling_on_sc=False,
                                             needs_layout_passes=False),
        scratch_types=[
            pltpu.VMEM((QPW * 3 // CH, CH), jnp.float32),
            pltpu.VMEM((QPW * 3 // CH, CH), jnp.int32),
            pltpu.VMEM((NCH_CAND, CH), jnp.int32),
            pltpu.VMEM((NCH_CAND, CH), jnp.int32),
            pltpu.VMEM((NCH_CAND, CH), jnp.int32),
            pltpu.VMEM((NCH_CAND, CH), jnp.int32),
            pltpu.VMEM((NCH_CAND, CH, 8), jnp.float32),
            pltpu.VMEM((NCH_CAND, CH, 8), jnp.float32),
            pltpu.VMEM((NCH_SEL, CH), jnp.int32),
            pltpu.VMEM((NCH_SEL, CH, CFEAT), jnp.float32),
            pltpu.VMEM((NCH_SEL, CH, 8), jnp.float32),
            pltpu.SMEM((G,), jnp.int32),
            pltpu.VMEM((G, 3 + CFEAT, NSAMPLE), jnp.float32),
            pltpu.VMEM((G, 3 + CFEAT, NSAMPLE), jnp.float32),
            pltpu.SemaphoreType.DMA,
            pltpu.SemaphoreType.DMA,
            pltpu.SemaphoreType.DMA,
            pltpu.SemaphoreType.DMA,
        ],
    )
    return run(qv, cv, rvf, xyzp, features)


_impl.__name__ = "kernel"
_JIT = None


def kernel(xyz, features, query_rv_xyz, query_rv_coords, rv_map):
    global _JIT
    if _JIT is None:
        _JIT = jax.jit(_impl)
    return _JIT(xyz, features, query_rv_xyz, query_rv_coords, rv_map)


# speedup vs baseline: 1.1337x; 1.1274x over previous
import functools

import jax
import jax.numpy as jnp
from jax import lax
from jax.experimental import pallas as pl
from jax.experimental.pallas import tpu as pltpu
from jax.experimental.pallas import tpu_sc as plsc

RADIUS2 = 4.0
NSAMPLE = 32
NCAND = 180
NCP = 192
M = 16384
CFEAT = 16
RV_H, RV_W, PPP = 64, 2048, 4

NCORES, NSUBC = 2, 16
NW = NCORES * NSUBC
QPW = M // NW
G = 16
NGRP = QPW // G
CH = 128
SH = 7
CMASK = CH - 1
NCH_CAND = (G * NCP) // CH
NCH_RV = (G * 48) // CH
NCH_SEL = (G * NSAMPLE) // CH
OW = (3 + CFEAT) * NSAMPLE


def _splat(x, dtype=jnp.int32):
    return jnp.full((16,), x, dtype=dtype)


def _vgather(v, idx):
    return v.at[idx].get(mode="promise_in_bounds")


def _elem(buf, pos):
    return plsc.load_gather(buf, [_splat(pos >> SH), _splat(pos & CMASK)])


def _sc_body(q_h, c_h, rv8_h, xyzp_h, feat_h, out_h,
             qbuf, cbuf, eidx0, eidx1, cr0, cr1, cf0, cf1, cxyz0, cxyz1,
             sel, gfeat, gxyz, cnts, outb0, outb1, semb, semc, seme, semg):
    wid = lax.axis_index("s") * NCORES + lax.axis_index("c")
    qbase = wid * QPW
    qrow = wid * (QPW * 3 // CH)
    iota = jnp.arange(16, dtype=jnp.int32)

    pltpu.sync_copy(q_h.at[pl.ds(qrow, QPW * 3 // CH)], qbuf)
    pltpu.sync_copy(c_h.at[pl.ds(qrow, QPW * 3 // CH)], cbuf)

    eidxs = (eidx0, eidx1)
    crs = (cr0, cr1)
    cfs = (cf0, cf1)
    cxyzs = (cxyz0, cxyz1)
    outbs = (outb0, outb1)

    def build_group(g, eidx):
        for i in range(G):
            lq = g * G + i
            rsp = _elem(cbuf, lq * 3 + 1) & jnp.int32(RV_H - 1)
            csp = _elem(cbuf, lq * 3 + 2) & jnp.int32(RV_W - 1)
            cells = []
            for jj in range(3):
                u = iota + 16 * jj
                oh = u // 9 - 2
                ow = 2 * (u % 9) - 8
                rr = jnp.clip(rsp + oh, 0, RV_H - 1)
                cc = (csp + ow) & jnp.int32(RV_W - 1)
                cells.append(rr * RV_W + cc)
            for jj in range(3):
                p = i * 48 + 16 * jj
                eidx[p >> SH, pl.ds(p & CMASK, 16)] = cells[jj] >> 1

    def fire_rv(eidx, cr):
        for j in range(NCH_RV):
            pltpu.make_async_copy(rv8_h.at[eidx.at[j]], cr.at[j],
                                  semb).start()

    def drain_rv(eidx, cr):
        for j in range(NCH_RV):
            pltpu.make_async_copy(rv8_h.at[eidx.at[j]], cr.at[j],
                                  semb).wait()

    def extract_group(g, cr, cf):
        @plsc.parallel_loop(0, G)
        def _extract(i):
            lq = g * G + i
            par4 = (_elem(cbuf, lq * 3 + 2) & 1) * 4
            for jj2 in range(12):
                t = _splat(16 * jj2) + iota
                gr = _splat(i * 48) + (t >> 2)
                cd = plsc.load_gather(
                    cr, [gr >> SH, gr & CMASK, par4 + (t & 3)])
                p = _splat(i * NCP + 16 * jj2) + iota
                plsc.store_scatter(cf, [p >> SH, p & CMASK], cd)


    def fire_xyz(cand, cxyz):
        for j in range(NCH_CAND):
            pltpu.make_async_copy(xyzp_h.at[cand.at[j]], cxyz.at[j],
                                  semc).start()

    def drain_xyz(cand, cxyz):
        for j in range(NCH_CAND):
            pltpu.make_async_copy(xyzp_h.at[cand.at[j]], cxyz.at[j],
                                  semc).wait()

    z16i = jnp.zeros((16,), jnp.int32)
    for j in range(NCH_SEL):
        for h8 in range(CH // 16):
            sel[j, pl.ds(16 * h8, 16)] = z16i

    build_group(0, eidx0)
    fire_rv(eidx0, cr0)
    drain_rv(eidx0, cr0)
    extract_group(0, cr0, cf0)
    fire_xyz(cf0, cxyz0)
    build_group(1, eidx1)
    fire_rv(eidx1, cr1)

    def pair_body(t, carry):
        for par in range(2):
            g = 2 * t + par
            eidx, cand, cxyz = eidxs[par], cfs[par], cxyzs[par]
            neidx, ncr, ncf, ncxyz = (eidxs[1 - par], crs[1 - par],
                                      cfs[1 - par], cxyzs[1 - par])

            drain_xyz(cand, cxyz)

            @pl.when(g < NGRP - 1)
            def _():
                drain_rv(neidx, ncr)
                extract_group(g + 1, ncr, ncf)
                fire_xyz(ncf, ncxyz)

            @pl.when(g < NGRP - 2)
            def _():
                build_group(g + 2, eidx)

            @plsc.parallel_loop(0, G)
            def select_body(i):
                lq = g * G + i
                xq = _elem(qbuf, lq * 3)
                yq = _elem(qbuf, lq * 3 + 1)
                zq = _elem(qbuf, lq * 3 + 2)
                z16 = _splat(0)
                cnt = jnp.int32(0)
                for jj in range(12):
                    p = _splat(i * NCP + 16 * jj) + iota
                    pr, pc = p >> SH, p & CMASK
                    cd = plsc.load_gather(cand, [pr, pc])
                    x = plsc.load_gather(cxyz, [pr, pc, z16])
                    y = plsc.load_gather(cxyz, [pr, pc, z16 + 1])
                    z = plsc.load_gather(cxyz, [pr, pc, z16 + 2])
                    dx, dy, dz = x - xq, y - yq, z - zq
                    d2 = dx * dx + dy * dy + dz * dz
                    val = d2 < RADIUS2
                    if jj == 11:
                        val = val & (iota < (NCAND - 16 * 11))
                    vi = val.astype(jnp.int32)
                    pref = plsc.cumsum(vi)
                    rank = cnt + pref - 1
                    m = val & (rank < NSAMPLE)
                    sp = _splat(i * NSAMPLE) + rank
                    plsc.store_scatter(sel, [sp >> SH, sp & CMASK], cd, mask=m)
                    cnt = cnt + pref[15]
                sp0 = i * NSAMPLE
                fsv = plsc.load_gather(
                    sel, [_splat(sp0 >> SH), _splat(sp0 & CMASK)])
                for h in range(2):
                    k = iota + 16 * h
                    spk = sp0 + k
                    cur = plsc.load_gather(sel, [spk >> SH, spk & CMASK])
                    new = jnp.where(k < cnt, cur, fsv)
                    plsc.store_scatter(sel, [spk >> SH, spk & CMASK], new)
                cnts[i] = cnt

            for j in range(NCH_SEL):
                pltpu.make_async_copy(feat_h.at[sel.at[j]],
                                      gfeat.at[j], seme).start()
                pltpu.make_async_copy(xyzp_h.at[sel.at[j]],
                                      gxyz.at[j], seme).start()

            @pl.when(g < NGRP - 2)
            def _():
                fire_rv(eidx, crs[par])

            for j in range(NCH_SEL):
                pltpu.make_async_copy(feat_h.at[sel.at[j]], gfeat.at[j],
                                      seme).wait()
                pltpu.make_async_copy(xyzp_h.at[sel.at[j]], gxyz.at[j],
                                      seme).wait()

            outb = outbs[par]
            @pl.when(g >= 2)
            def _():
                pltpu.make_async_copy(
                    outb, out_h.at[pl.ds(qbase + (g - 2) * G, G)],
                    semg).wait()

            @plsc.parallel_loop(0, G)
            def out_body(i):
                lq = g * G + i
                xq = _elem(qbuf, lq * 3)
                yq = _elem(qbuf, lq * 3 + 1)
                zq = _elem(qbuf, lq * 3 + 2)
                isp = _splat(0) + i
                qs = (xq, yq, zq)
                for h in range(2):
                    sp = _splat(i * NSAMPLE + 16 * h) + iota
                    sr, sc = sp >> SH, sp & CMASK
                    k = _splat(16 * h) + iota
                    for c in range(3 + CFEAT):
                        if c < 3:
                            v = plsc.load_gather(
                                gxyz, [sr, sc, _splat(c)]) - qs[c]
                        else:
                            v = plsc.load_gather(
                                gfeat, [sr, sc, _splat(c - 3)])
                        plsc.store_scatter(outb, [isp, _splat(c), k], v)

                @pl.when(cnts[i] == 0)
                def _():
                    z16f = jnp.zeros((16,), jnp.float32)
                    for h in range(2):
                        k = _splat(16 * h) + iota
                        for c in range(3 + CFEAT):
                            plsc.store_scatter(
                                outb, [isp, _splat(c), k], z16f)

            pltpu.make_async_copy(
                outb, out_h.at[pl.ds(qbase + g * G, G)], semg).start()
        return carry

    lax.fori_loop(0, NGRP // 2, pair_body, 0)

    for gl in (NGRP - 2, NGRP - 1):
        pltpu.make_async_copy(
            outbs[gl % 2], out_h.at[pl.ds(qbase + gl * G, G)],
            semg).wait()


def _impl(xyz, features, query_rv_xyz, query_rv_coords, rv_map):
    xyzp = jnp.concatenate(
        [xyz, jnp.zeros((xyz.shape[0], 5), jnp.float32)], axis=1)
    rv8 = rv_map.reshape(-1).reshape(524288 // 8, 8)
    qv = query_rv_xyz.reshape(M * 3 // CH, CH)
    cv = query_rv_coords.reshape(M * 3 // CH, CH)

    mesh = plsc.VectorSubcoreMesh(core_axis_name="c", subcore_axis_name="s",
                                  num_cores=NCORES, num_subcores=NSUBC)
    run = pl.kernel(
        _sc_body,
        out_type=jax.ShapeDtypeStruct((M, 3 + CFEAT, NSAMPLE),
                                      jnp.float32),
        mesh=mesh,
        compiler_params=pltpu.CompilerParams(use_tc_tiling_on_sc=False,
                                             needs_layout_passes=False),
        scratch_types=[
            pltpu.VMEM((QPW * 3 // CH, CH), jnp.float32),
            pltpu.VMEM((QPW * 3 // CH, CH), jnp.int32),
            pltpu.VMEM((NCH_RV, CH), jnp.int32),
            pltpu.VMEM((NCH_RV, CH), jnp.int32),
            pltpu.VMEM((NCH_RV, CH, 8), jnp.int32),
            pltpu.VMEM((NCH_RV, CH, 8), jnp.int32),
            pltpu.VMEM((NCH_CAND, CH), jnp.int32),
            pltpu.VMEM((NCH_CAND, CH), jnp.int32),
            pltpu.VMEM((NCH_CAND, CH, 8), jnp.float32),
            pltpu.VMEM((NCH_CAND, CH, 8), jnp.float32),
            pltpu.VMEM((NCH_SEL, CH), jnp.int32),
            pltpu.VMEM((NCH_SEL, CH, CFEAT), jnp.float32),
            pltpu.VMEM((NCH_SEL, CH, 8), jnp.float32),
            pltpu.SMEM((G,), jnp.int32),
            pltpu.VMEM((G, 3 + CFEAT, NSAMPLE), jnp.float32),
            pltpu.VMEM((G, 3 + CFEAT, NSAMPLE), jnp.float32),
            pltpu.SemaphoreType.DMA,
            pltpu.SemaphoreType.DMA,
            pltpu.SemaphoreType.DMA,
            pltpu.SemaphoreType.DMA,
        ],
    )
    return run(qv, cv, rv8, xyzp, features)


_impl.__name__ = "kernel"
_JIT = None


def kernel(xyz, features, query_rv_xyz, query_rv_coords, rv_map):
    global _JIT
    if _JIT is None:
        _JIT = jax.jit(_impl)
    return _JIT(xyz, features, query_rv_xyz, query_rv_coords, rv_map)


# speedup vs baseline: 1.1393x; 1.0049x over previous
import functools

import jax
import jax.numpy as jnp
from jax import lax
from jax.experimental import pallas as pl
from jax.experimental.pallas import tpu as pltpu
from jax.experimental.pallas import tpu_sc as plsc

RADIUS2 = 4.0
NSAMPLE = 32
NCAND = 180
NCP = 192
M = 16384
CFEAT = 16
RV_H, RV_W, PPP = 64, 2048, 4

NCORES, NSUBC = 2, 16
NW = NCORES * NSUBC
QPW = M // NW
G = 16
NGRP = QPW // G
CH = 128
SH = 7
CMASK = CH - 1
NCH_CAND = (G * NCP) // CH
NCH_RV = (G * 48) // CH
NCH_SEL = (G * NSAMPLE) // CH
OW = (3 + CFEAT) * NSAMPLE


def _splat(x, dtype=jnp.int32):
    return jnp.full((16,), x, dtype=dtype)


def _vgather(v, idx):
    return v.at[idx].get(mode="promise_in_bounds")


def _elem(buf, pos):
    return plsc.load_gather(buf, [_splat(pos >> SH), _splat(pos & CMASK)])


def _sc_body(q_h, c_h, rv8_h, xyzp_h, feat_h, out_h,
             qbuf, cbuf, eidx0, eidx1, cr0, cr1, cf0, cf1, cxyz0, cxyz1,
             sel, selpos, gfeat, cnts, outb0, outb1, semb, semc, seme, semg):
    wid = lax.axis_index("s") * NCORES + lax.axis_index("c")
    qbase = wid * QPW
    qrow = wid * (QPW * 3 // CH)
    iota = jnp.arange(16, dtype=jnp.int32)

    pltpu.sync_copy(q_h.at[pl.ds(qrow, QPW * 3 // CH)], qbuf)
    pltpu.sync_copy(c_h.at[pl.ds(qrow, QPW * 3 // CH)], cbuf)

    eidxs = (eidx0, eidx1)
    crs = (cr0, cr1)
    cfs = (cf0, cf1)
    cxyzs = (cxyz0, cxyz1)
    outbs = (outb0, outb1)

    def build_group(g, eidx):
        for i in range(G):
            lq = g * G + i
            rsp = _elem(cbuf, lq * 3 + 1) & jnp.int32(RV_H - 1)
            csp = _elem(cbuf, lq * 3 + 2) & jnp.int32(RV_W - 1)
            cells = []
            for jj in range(3):
                u = iota + 16 * jj
                oh = u // 9 - 2
                ow = 2 * (u % 9) - 8
                rr = jnp.clip(rsp + oh, 0, RV_H - 1)
                cc = (csp + ow) & jnp.int32(RV_W - 1)
                cells.append(rr * RV_W + cc)
            for jj in range(3):
                p = i * 48 + 16 * jj
                eidx[p >> SH, pl.ds(p & CMASK, 16)] = cells[jj] >> 1

    def fire_rv(eidx, cr):
        for j in range(NCH_RV):
            pltpu.make_async_copy(rv8_h.at[eidx.at[j]], cr.at[j],
                                  semb).start()

    def drain_rv(eidx, cr):
        for j in range(NCH_RV):
            pltpu.make_async_copy(rv8_h.at[eidx.at[j]], cr.at[j],
                                  semb).wait()

    def extract_group(g, cr, cf):
        @plsc.parallel_loop(0, G)
        def _extract(i):
            lq = g * G + i
            par4 = (_elem(cbuf, lq * 3 + 2) & 1) * 4
            for jj2 in range(12):
                t = _splat(16 * jj2) + iota
                gr = _splat(i * 48) + (t >> 2)
                cd = plsc.load_gather(
                    cr, [gr >> SH, gr & CMASK, par4 + (t & 3)])
                p = _splat(i * NCP + 16 * jj2) + iota
                plsc.store_scatter(cf, [p >> SH, p & CMASK], cd)


    def fire_xyz(cand, cxyz):
        for j in range(NCH_CAND):
            pltpu.make_async_copy(xyzp_h.at[cand.at[j]], cxyz.at[j],
                                  semc).start()

    def drain_xyz(cand, cxyz):
        for j in range(NCH_CAND):
            pltpu.make_async_copy(xyzp_h.at[cand.at[j]], cxyz.at[j],
                                  semc).wait()

    z16i = jnp.zeros((16,), jnp.int32)
    for j in range(NCH_SEL):
        for h8 in range(CH // 16):
            sel[j, pl.ds(16 * h8, 16)] = z16i
            selpos[j, pl.ds(16 * h8, 16)] = z16i

    build_group(0, eidx0)
    fire_rv(eidx0, cr0)
    drain_rv(eidx0, cr0)
    extract_group(0, cr0, cf0)
    fire_xyz(cf0, cxyz0)
    build_group(1, eidx1)
    fire_rv(eidx1, cr1)

    def pair_body(t, carry):
        for par in range(2):
            g = 2 * t + par
            eidx, cand, cxyz = eidxs[par], cfs[par], cxyzs[par]
            neidx, ncr, ncf, ncxyz = (eidxs[1 - par], crs[1 - par],
                                      cfs[1 - par], cxyzs[1 - par])

            drain_xyz(cand, cxyz)

            @pl.when(g < NGRP - 1)
            def _():
                drain_rv(neidx, ncr)
                extract_group(g + 1, ncr, ncf)
                fire_xyz(ncf, ncxyz)

            @pl.when(g < NGRP - 2)
            def _():
                build_group(g + 2, eidx)

            @plsc.parallel_loop(0, G)
            def select_body(i):
                lq = g * G + i
                xq = _elem(qbuf, lq * 3)
                yq = _elem(qbuf, lq * 3 + 1)
                zq = _elem(qbuf, lq * 3 + 2)
                z16 = _splat(0)
                cnt = jnp.int32(0)
                for jj in range(12):
                    p = _splat(i * NCP + 16 * jj) + iota
                    pr, pc = p >> SH, p & CMASK
                    cd = plsc.load_gather(cand, [pr, pc])
                    x = plsc.load_gather(cxyz, [pr, pc, z16])
                    y = plsc.load_gather(cxyz, [pr, pc, z16 + 1])
                    z = plsc.load_gather(cxyz, [pr, pc, z16 + 2])
                    dx, dy, dz = x - xq, y - yq, z - zq
                    d2 = dx * dx + dy * dy + dz * dz
                    val = d2 < RADIUS2
                    if jj == 11:
                        val = val & (iota < (NCAND - 16 * 11))
                    vi = val.astype(jnp.int32)
                    pref = plsc.cumsum(vi)
                    rank = cnt + pref - 1
                    m = val & (rank < NSAMPLE)
                    sp = _splat(i * NSAMPLE) + rank
                    plsc.store_scatter(sel, [sp >> SH, sp & CMASK], cd, mask=m)
                    plsc.store_scatter(selpos, [sp >> SH, sp & CMASK], p,
                                       mask=m)
                    cnt = cnt + pref[15]
                sp0 = i * NSAMPLE
                fsv = plsc.load_gather(
                    sel, [_splat(sp0 >> SH), _splat(sp0 & CMASK)])
                fpv = plsc.load_gather(
                    selpos, [_splat(sp0 >> SH), _splat(sp0 & CMASK)])
                for h in range(2):
                    k = iota + 16 * h
                    spk = sp0 + k
                    cur = plsc.load_gather(sel, [spk >> SH, spk & CMASK])
                    new = jnp.where(k < cnt, cur, fsv)
                    plsc.store_scatter(sel, [spk >> SH, spk & CMASK], new)
                    curp = plsc.load_gather(selpos, [spk >> SH, spk & CMASK])
                    newp = jnp.where(k < cnt, curp, fpv)
                    plsc.store_scatter(selpos, [spk >> SH, spk & CMASK], newp)
                cnts[i] = cnt

            for j in range(NCH_SEL):
                pltpu.make_async_copy(feat_h.at[sel.at[j]],
                                      gfeat.at[j], seme).start()

            @pl.when(g < NGRP - 2)
            def _():
                fire_rv(eidx, crs[par])

            for j in range(NCH_SEL):
                pltpu.make_async_copy(feat_h.at[sel.at[j]], gfeat.at[j],
                                      seme).wait()

            outb = outbs[par]
            @pl.when(g >= 2)
            def _():
                pltpu.make_async_copy(
                    outb, out_h.at[pl.ds(qbase + (g - 2) * G, G)],
                    semg).wait()

            @plsc.parallel_loop(0, G)
            def out_body(i):
                lq = g * G + i
                xq = _elem(qbuf, lq * 3)
                yq = _elem(qbuf, lq * 3 + 1)
                zq = _elem(qbuf, lq * 3 + 2)
                isp = _splat(0) + i
                qs = (xq, yq, zq)
                for h in range(2):
                    sp = _splat(i * NSAMPLE + 16 * h) + iota
                    sr, sc = sp >> SH, sp & CMASK
                    k = _splat(16 * h) + iota
                    pos = plsc.load_gather(selpos, [sr, sc])
                    posr, posc = pos >> SH, pos & CMASK
                    for c in range(3 + CFEAT):
                        if c < 3:
                            v = plsc.load_gather(
                                cxyz, [posr, posc, _splat(c)]) - qs[c]
                        else:
                            v = plsc.load_gather(
                                gfeat, [sr, sc, _splat(c - 3)])
                        plsc.store_scatter(outb, [isp, _splat(c), k], v)

                @pl.when(cnts[i] == 0)
                def _():
                    z16f = jnp.zeros((16,), jnp.float32)
                    for h in range(2):
                        k = _splat(16 * h) + iota
                        for c in range(3 + CFEAT):
                            plsc.store_scatter(
                                outb, [isp, _splat(c), k], z16f)

            pltpu.make_async_copy(
                outb, out_h.at[pl.ds(qbase + g * G, G)], semg).start()
        return carry

    lax.fori_loop(0, NGRP // 2, pair_body, 0)

    for gl in (NGRP - 2, NGRP - 1):
        pltpu.make_async_copy(
            outbs[gl % 2], out_h.at[pl.ds(qbase + gl * G, G)],
            semg).wait()


def _impl(xyz, features, query_rv_xyz, query_rv_coords, rv_map):
    xyzp = jnp.concatenate(
        [xyz, jnp.zeros((xyz.shape[0], 5), jnp.float32)], axis=1)
    rv8 = rv_map.reshape(-1).reshape(524288 // 8, 8)
    qv = query_rv_xyz.reshape(M * 3 // CH, CH)
    cv = query_rv_coords.reshape(M * 3 // CH, CH)

    mesh = plsc.VectorSubcoreMesh(core_axis_name="c", subcore_axis_name="s",
                                  num_cores=NCORES, num_subcores=NSUBC)
    run = pl.kernel(
        _sc_body,
        out_type=jax.ShapeDtypeStruct((M, 3 + CFEAT, NSAMPLE),
                                      jnp.float32),
        mesh=mesh,
        compiler_params=pltpu.CompilerParams(use_tc_tiling_on_sc=False,
                                             needs_layout_passes=False),
        scratch_types=[
            pltpu.VMEM((QPW * 3 // CH, CH), jnp.float32),
            pltpu.VMEM((QPW * 3 // CH, CH), jnp.int32),
            pltpu.VMEM((NCH_RV, CH), jnp.int32),
            pltpu.VMEM((NCH_RV, CH), jnp.int32),
            pltpu.VMEM((NCH_RV, CH, 8), jnp.int32),
            pltpu.VMEM((NCH_RV, CH, 8), jnp.int32),
            pltpu.VMEM((NCH_CAND, CH), jnp.int32),
            pltpu.VMEM((NCH_CAND, CH), jnp.int32),
            pltpu.VMEM((NCH_CAND, CH, 8), jnp.float32),
            pltpu.VMEM((NCH_CAND, CH, 8), jnp.float32),
            pltpu.VMEM((NCH_SEL, CH), jnp.int32),
            pltpu.VMEM((NCH_SEL, CH), jnp.int32),
            pltpu.VMEM((NCH_SEL, CH, CFEAT), jnp.float32),
            pltpu.SMEM((G,), jnp.int32),
            pltpu.VMEM((G, 3 + CFEAT, NSAMPLE), jnp.float32),
            pltpu.VMEM((G, 3 + CFEAT, NSAMPLE), jnp.float32),
            pltpu.SemaphoreType.DMA,
            pltpu.SemaphoreType.DMA,
            pltpu.SemaphoreType.DMA,
            pltpu.SemaphoreType.DMA,
        ],
    )
    return run(qv, cv, rv8, xyzp, features)


_impl.__name__ = "kernel"
_JIT = None


def kernel(xyz, features, query_rv_xyz, query_rv_coords, rv_map):
    global _JIT
    if _JIT is None:
        _JIT = jax.jit(_impl)
    return _JIT(xyz, features, query_rv_xyz, query_rv_coords, rv_map)
